# Initial kernel scaffold; baseline (speedup 1.0000x reference)
import jax
import jax.numpy as jnp
from jax.experimental import pallas as pl


def kernel(A_indices, A_values, L_indices, L_values, embed, W1, W2, w_s):
    raise NotImplementedError("write your pallas kernel here")


# speedup vs baseline: 25.5163x; 25.5163x over previous
import functools

import jax
import jax.numpy as jnp
from jax import lax
from jax.experimental import pallas as pl
from jax.experimental.pallas import tpu as pltpu
from jax.experimental.pallas import tpu_sc as plsc

N = 50000
E = 800000
EL = E + N
D = 64
ITERS = 10
TAU = 0.5

A_CH = 128
A_CHUNKS_PER_TILE = 392
A_NCH = 16 * A_CHUNKS_PER_TILE
E_PAD = A_NCH * A_CH
HALF = N // 2
ACC_ROWS = 25088

L_CH = 512
L_CHUNKS_PER_TILE = 104
L_NCH = 16 * L_CHUNKS_PER_TILE
EL_PAD = L_NCH * L_CH
NPAD = 50176
SLICE = NPAD // 16

_mesh = plsc.VectorSubcoreMesh(core_axis_name="c", subcore_axis_name="s")


def _prep_edges(indices, values, e_pad, nch, ch, sub, pad_row):
    e = values.shape[0]
    rows = jnp.concatenate(
        [indices[0], jnp.full((e_pad - e,), pad_row, jnp.int32)])
    cols = jnp.concatenate([indices[1], jnp.zeros((e_pad - e,), jnp.int32)])
    vals = jnp.concatenate([values, jnp.zeros((e_pad - e,), jnp.float32)])
    rc = jnp.stack([rows.reshape(nch, sub, ch // sub),
                    cols.reshape(nch, sub, ch // sub)], axis=1)
    return rc, vals.reshape(nch, ch)


@functools.partial(
    pl.kernel,
    out_type=jax.ShapeDtypeStruct((N, D), jnp.float32),
    mesh=_mesh,
    compiler_params=pltpu.CompilerParams(use_tc_tiling_on_sc=False,
                                         needs_layout_passes=False),
    scratch_types=dict(
        rc0=pltpu.VMEM((2, 1, A_CH), jnp.int32),
        rc1=pltpu.VMEM((2, 1, A_CH), jnp.int32),
        val0=pltpu.VMEM((A_CH,), jnp.float32),
        val1=pltpu.VMEM((A_CH,), jnp.float32),
        idx0=pltpu.VMEM((A_CH,), jnp.int32),
        idx1=pltpu.VMEM((A_CH,), jnp.int32),
        rows0=pltpu.VMEM((A_CH, D), jnp.float32),
        rows1=pltpu.VMEM((A_CH, D), jnp.float32),
        zb=pltpu.VMEM((A_CH, D), jnp.float32),
        acc=pltpu.VMEM_SHARED((ACC_ROWS, D), jnp.float32),
        s_rc0=pltpu.SemaphoreType.DMA,
        s_rc1=pltpu.SemaphoreType.DMA,
        s_v0=pltpu.SemaphoreType.DMA,
        s_v1=pltpu.SemaphoreType.DMA,
        s_g0=pltpu.SemaphoreType.DMA,
        s_g1=pltpu.SemaphoreType.DMA,
        s_c0=pltpu.SemaphoreType.DMA,
        s_c1=pltpu.SemaphoreType.DMA,
    ),
)
def _spmm_kernel(a_rc, a_val, x, out, *, rc0, rc1, val0, val1, idx0, idx1,
                 rows0, rows1, zb, acc, s_rc0, s_rc1, s_v0, s_v1, s_g0, s_g1,
                 s_c0, s_c1):
    c = lax.axis_index("c")
    s = lax.axis_index("s")
    rc = (rc0, rc1)
    val = (val0, val1)
    idx = (idx0, idx1)
    rows = (rows0, rows1)
    s_rc = (s_rc0, s_rc1)
    s_v = (s_v0, s_v1)
    s_g = (s_g0, s_g1)
    s_c = (s_c0, s_c1)
    nk = A_CHUNKS_PER_TILE
    zero16 = jnp.zeros((16,), jnp.float32)
    row_base = c * HALF

    def _zb_body(i, _):
        for q in range(4):
            zb[i, pl.ds(16 * q, 16)] = zero16
        return 0
    lax.fori_loop(0, A_CH, _zb_body, 0)
    for kk in range(ACC_ROWS // A_CH // 16 + 1):
        blk = s + 16 * kk

        @pl.when(blk < ACC_ROWS // A_CH)
        def _():
            pltpu.sync_copy(zb, acc.at[pl.ds(blk * A_CH, A_CH)])
    plsc.subcore_barrier()

    def _chunk(j):
        return s + 16 * j

    def _start_edges(j, b):
        pltpu.async_copy(a_rc.at[_chunk(j)], rc[b], s_rc[b])

    def _start_vals(j, b):
        pltpu.async_copy(a_val.at[_chunk(j)], val[b], s_v[b])

    def _wait_edges(b):
        pltpu.make_async_copy(a_rc.at[0], rc[b], s_rc[b]).wait()

    def _wait_vals(b):
        pltpu.make_async_copy(a_val.at[0], val[b], s_v[b]).wait()

    def _start_gather(b):
        pltpu.async_copy(x.at[rc[b].at[1, 0]], rows[b], s_g[b])

    def _wait_gather(b):
        pltpu.make_async_copy(x.at[rc[b].at[1, 0]], rows[b], s_g[b]).wait()

    def _start_scatter(b):
        pltpu.async_copy(rows[b], acc.at[idx[b]], s_c[b], add=True)

    def _wait_scatter(b):
        pltpu.make_async_copy(rows[b], acc.at[idx[b]], s_c[b]).wait()

    _start_edges(0, 0)
    _start_vals(0, 0)
    _start_edges(1, 1)
    _start_vals(1, 1)
    _wait_edges(0)
    _start_gather(0)

    def _iter(jj, _):
        for b in (0, 1):
            j = 2 * jj + b
            nxt = 1 - b

            @pl.when(j + 1 < nk)
            def _():
                _wait_edges(nxt)

            @pl.when(j >= 1)
            def _():
                _wait_scatter(nxt)

            @pl.when(j + 1 < nk)
            def _():
                _start_gather(nxt)

            _wait_gather(b)
            for q in range(A_CH // 16):
                r = rc[b][0, 0, pl.ds(16 * q, 16)]
                local = r - row_base
                owned = (local >= 0) & (local < HALF)
                dummy = HALF + (local & 63)
                idx[b][pl.ds(16 * q, 16)] = jnp.where(owned, local, dummy)

            @pl.when(j + 2 < nk)
            def _():
                _start_edges(j + 2, b)

            _wait_vals(b)

            def _scale(i, _):
                valv = val[b][pl.ds(16 * i, 16)]
                for u in range(16):
                    v = valv[u]
                    e = 16 * i + u
                    for q in range(4):
                        rows[b][e, pl.ds(16 * q, 16)] = (
                            rows[b][e, pl.ds(16 * q, 16)] * v)
                return 0
            lax.fori_loop(0, A_CH // 16, _scale, 0)

            @pl.when(j + 2 < nk)
            def _():
                _start_vals(j + 2, b)

            _start_scatter(b)
        return 0

    lax.fori_loop(0, nk // 2, _iter, 0)
    _wait_scatter(1)
    plsc.subcore_barrier()

    nfull = HALF // A_CH
    rem = HALF - nfull * A_CH
    for kk in range(nfull // 16 + 1):
        blk = s + 16 * kk

        @pl.when(blk < nfull)
        def _():
            pltpu.sync_copy(acc.at[pl.ds(blk * A_CH, A_CH)],
                            out.at[pl.ds(row_base + blk * A_CH, A_CH)])

        @pl.when(blk == nfull)
        def _():
            pltpu.sync_copy(acc.at[pl.ds(nfull * A_CH, rem)],
                            out.at[pl.ds(row_base + nfull * A_CH, rem)])


_TC_BLK = 2048
_TC_GRID = (N + _TC_BLK - 1) // _TC_BLK


def _tc_layer(ax, x, w):
    def body(ax_ref, x_ref, w_ref, o_ref):
        h = jnp.dot(ax_ref[...], w_ref[...].T,
                    preferred_element_type=jnp.float32)
        o_ref[...] = x_ref[...] + jnp.maximum(h, 0.0)

    return pl.pallas_call(
        body,
        grid=(_TC_GRID,),
        in_specs=[
            pl.BlockSpec((_TC_BLK, D), lambda i: (i, 0)),
            pl.BlockSpec((_TC_BLK, D), lambda i: (i, 0)),
            pl.BlockSpec((D, D), lambda i: (0, 0)),
        ],
        out_specs=pl.BlockSpec((_TC_BLK, D), lambda i: (i, 0)),
        out_shape=jax.ShapeDtypeStruct((N, D), jnp.float32),
    )(ax, x, w)


def _tc_layer_score(ax, x, w, ws):
    def body(ax_ref, x_ref, w_ref, ws_ref, o_ref):
        h = jnp.dot(ax_ref[...], w_ref[...].T,
                    preferred_element_type=jnp.float32)
        xf = x_ref[...] + jnp.maximum(h, 0.0)
        o_ref[...] = jnp.dot(xf.astype(jnp.bfloat16),
                             ws_ref[...].astype(jnp.bfloat16).T,
                             preferred_element_type=jnp.float32)

    return pl.pallas_call(
        body,
        grid=(_TC_GRID,),
        in_specs=[
            pl.BlockSpec((_TC_BLK, D), lambda i: (i, 0)),
            pl.BlockSpec((_TC_BLK, D), lambda i: (i, 0)),
            pl.BlockSpec((D, D), lambda i: (0, 0)),
            pl.BlockSpec((1, D), lambda i: (0, 0)),
        ],
        out_specs=pl.BlockSpec((_TC_BLK, 1), lambda i: (i, 0)),
        out_shape=jax.ShapeDtypeStruct((N, 1), jnp.float32),
    )(ax, x, w, ws)


@functools.partial(
    pl.kernel,
    out_type=jax.ShapeDtypeStruct((N,), jnp.float32),
    mesh=_mesh,
    compiler_params=pltpu.CompilerParams(use_tc_tiling_on_sc=False,
                                         needs_layout_passes=False),
    scratch_types=dict(
        rc0=pltpu.VMEM((2, 4, 128), jnp.int32),
        rc1=pltpu.VMEM((2, 4, 128), jnp.int32),
        val0=pltpu.VMEM((L_CH,), jnp.float32),
        val1=pltpu.VMEM((L_CH,), jnp.float32),
        vg=pltpu.VMEM((L_CH,), jnp.float32),
        wbuf=pltpu.VMEM((SLICE,), jnp.float32),
        zb=pltpu.VMEM((SLICE,), jnp.float32),
        p16=pltpu.VMEM((16,), jnp.float32),
        ppart=pltpu.VMEM((16, 16), jnp.float32),
        vsh=pltpu.VMEM_SHARED((NPAD,), jnp.float32),
        acc=pltpu.VMEM_SHARED((NPAD,), jnp.float32),
        parts=pltpu.VMEM_SHARED((16, 16), jnp.float32),
        s_rc0=pltpu.SemaphoreType.DMA,
        s_rc1=pltpu.SemaphoreType.DMA,
        s_v0=pltpu.SemaphoreType.DMA,
        s_v1=pltpu.SemaphoreType.DMA,
        s_g=pltpu.SemaphoreType.DMA,
        s_c=pltpu.SemaphoreType.DMA,
    ),
)
def _fiedler_kernel(l_rc, l_val, s_in, out, *, rc0, rc1, val0, val1, vg, wbuf,
                    zb, p16, ppart, vsh, acc, parts, s_rc0, s_rc1, s_v0, s_v1,
                    s_g, s_c):
    c = lax.axis_index("c")
    t = lax.axis_index("s")

    @pl.when(c == 0)
    def _core0():
        rc = (rc0, rc1)
        val = (val0, val1)
        s_rc = (s_rc0, s_rc1)
        s_v = (s_v0, s_v1)
        nk = L_CHUNKS_PER_TILE
        zero16 = jnp.zeros((16,), jnp.float32)

        def _zb_body(i, _):
            zb[pl.ds(16 * i, 16)] = zero16
            return 0
        lax.fori_loop(0, SLICE // 16, _zb_body, 0)

        base = t * SLICE

        @pl.when(t < 15)
        def _():
            pltpu.sync_copy(s_in.at[pl.ds(base, SLICE)],
                            vsh.at[pl.ds(base, SLICE)])

        @pl.when(t == 15)
        def _():
            pltpu.sync_copy(s_in.at[pl.ds(15 * SLICE, N - 15 * SLICE)],
                            vsh.at[pl.ds(15 * SLICE, N - 15 * SLICE)])
            pltpu.sync_copy(zb.at[pl.ds(0, NPAD - N)],
                            vsh.at[pl.ds(N, NPAD - N)])
        pltpu.sync_copy(zb, acc.at[pl.ds(base, SLICE)])
        plsc.subcore_barrier()

        def _start_edges(j, b):
            ck = t * nk + j
            pltpu.async_copy(l_rc.at[ck], rc[b], s_rc[b])
            pltpu.async_copy(l_val.at[ck], val[b], s_v[b])

        def _wait_edges(b):
            pltpu.make_async_copy(l_rc.at[0], rc[b], s_rc[b]).wait()
            pltpu.make_async_copy(l_val.at[0], val[b], s_v[b]).wait()

        def _one_iter(it, _):
            _start_edges(0, 0)
            _start_edges(1, 1)

            def _edge_chunk(jj, _):
                for b in (0, 1):
                    j = 2 * jj + b
                    _wait_edges(b)
                    gd = []
                    for i in range(4):
                        gd.append(pltpu.async_copy(
                            vsh.at[rc[b].at[1, i]],
                            vg.at[pl.ds(128 * i, 128)], s_g))
                    for d in gd:
                        d.wait()
                    for q in range(L_CH // 16):
                        vg[pl.ds(16 * q, 16)] = (
                            vg[pl.ds(16 * q, 16)] * val[b][pl.ds(16 * q, 16)])
                    cd = []
                    for i in range(4):
                        cd.append(pltpu.async_copy(
                            vg.at[pl.ds(128 * i, 128)],
                            acc.at[rc[b].at[0, i]], s_c, add=True))
                    for d in cd:
                        d.wait()

                    @pl.when(j + 2 < nk)
                    def _():
                        _start_edges(j + 2, b)
                return 0

            lax.fori_loop(0, nk // 2, _edge_chunk, 0)
            plsc.subcore_barrier()

            pltpu.sync_copy(acc.at[pl.ds(base, SLICE)], wbuf)
            pltpu.sync_copy(zb, acc.at[pl.ds(base, SLICE)])

            def _shrink(q, sq):
                a = wbuf[pl.ds(16 * q, 16)]
                w = a - TAU * jnp.sign(a)
                wbuf[pl.ds(16 * q, 16)] = w
                return sq + w * w
            sq = lax.fori_loop(0, SLICE // 16, _shrink,
                               jnp.zeros((16,), jnp.float32))
            p16[...] = sq
            pltpu.sync_copy(p16, parts.at[t])
            plsc.subcore_barrier()

            pltpu.sync_copy(parts, ppart)
            tot = jnp.zeros((16,), jnp.float32)
            for i in range(16):
                tot = tot + ppart[i, pl.ds(0, 16)]
            ss = jnp.full((16,), jnp.sum(tot), jnp.float32)
            ss = jnp.maximum(ss, 1e-24)
            bits = lax.bitcast_convert_type(ss, jnp.int32)
            y = lax.bitcast_convert_type(
                jnp.int32(0x5F3759DF) - lax.shift_right_logical(bits, 1),
                jnp.float32)
            for _n in range(3):
                y = y * (1.5 - 0.5 * ss * y * y)

            def _scale(q, _):
                wbuf[pl.ds(16 * q, 16)] = wbuf[pl.ds(16 * q, 16)] * y
                return 0
            lax.fori_loop(0, SLICE // 16, _scale, 0)
            pltpu.sync_copy(wbuf, vsh.at[pl.ds(base, SLICE)])

            @pl.when(it == ITERS - 1)
            def _():
                @pl.when(t < 15)
                def _():
                    pltpu.sync_copy(wbuf, out.at[pl.ds(base, SLICE)])

                @pl.when(t == 15)
                def _():
                    pltpu.sync_copy(
                        wbuf.at[pl.ds(0, N - 15 * SLICE)],
                        out.at[pl.ds(15 * SLICE, N - 15 * SLICE)])
            plsc.subcore_barrier()
            return 0

        lax.fori_loop(0, ITERS, _one_iter, 0)


def kernel(A_indices, A_values, L_indices, L_values, embed, W1, W2, w_s):
    a_rc, a_val = _prep_edges(A_indices, A_values, E_PAD, A_NCH, A_CH, 1,
                              pad_row=N + 100)
    l_rc, l_val = _prep_edges(L_indices, L_values, EL_PAD, L_NCH, L_CH, 4,
                              pad_row=0)
    x0 = embed
    ax0 = _spmm_kernel(a_rc, a_val, x0)
    x1 = _tc_layer(ax0, x0, W1)
    ax1 = _spmm_kernel(a_rc, a_val, x1)
    s = _tc_layer_score(ax1, x1, W2, w_s)
    return _fiedler_kernel(l_rc, l_val, s[:, 0])
